# Initial kernel scaffold; baseline (speedup 1.0000x reference)
import jax
import jax.numpy as jnp
from jax.experimental import pallas as pl


def kernel(x):
    raise NotImplementedError("write your pallas kernel here")


# speedup vs baseline: 3.1102x; 3.1102x over previous
import jax
import jax.numpy as jnp
from jax.experimental import pallas as pl
from jax.experimental.pallas import tpu as pltpu

_SINKHORN_ITERS = 5
_TAU = 0.01
_EPS = 1e-6


def _sinkhorn_body(x_ref, o_ref):
    y = x_ref[0] * (1.0 / _TAU)
    for _ in range(_SINKHORN_ITERS):
        m = jnp.max(y, axis=1, keepdims=True)
        s = jnp.sum(jnp.exp(y - m), axis=1, keepdims=True)
        y = y - (jnp.log(s) + m)
        m = jnp.max(y, axis=0, keepdims=True)
        s = jnp.sum(jnp.exp(y - m), axis=0, keepdims=True)
        y = y - (jnp.log(s) + m)
    o_ref[0] = jnp.exp(y) + _EPS


def kernel(x):
    b, n, _ = x.shape
    return pl.pallas_call(
        _sinkhorn_body,
        grid=(b,),
        in_specs=[pl.BlockSpec((1, n, n), lambda i: (i, 0, 0))],
        out_specs=pl.BlockSpec((1, n, n), lambda i: (i, 0, 0)),
        out_shape=jax.ShapeDtypeStruct(x.shape, x.dtype),
        compiler_params=pltpu.CompilerParams(
            dimension_semantics=("parallel",),
        ),
    )(x)


# speedup vs baseline: 3.3550x; 1.0787x over previous
import jax
import jax.numpy as jnp
from jax.experimental import pallas as pl
from jax.experimental.pallas import tpu as pltpu

_SINKHORN_ITERS = 5
_TAU = 0.01
_EPS = 1e-6
_LOG2E = 1.4426950408889634


def _sinkhorn_body(x_ref, o_ref):
    y0 = x_ref[0] * (_LOG2E / _TAU)

    m = jnp.max(y0, axis=1, keepdims=True)
    s = jnp.sum(jnp.exp2(y0 - m), axis=1, keepdims=True)
    r = m + jnp.log2(s)

    for it in range(_SINKHORN_ITERS):
        u = y0 - r
        m = jnp.max(u, axis=0, keepdims=True)
        e = jnp.exp2(u - m)
        s = jnp.sum(e, axis=0, keepdims=True)
        if it == _SINKHORN_ITERS - 1:
            o_ref[0] = e * (1.0 / s) + _EPS
            break
        c = m + jnp.log2(s)
        t = y0 - c
        m = jnp.max(t, axis=1, keepdims=True)
        s = jnp.sum(jnp.exp2(t - m), axis=1, keepdims=True)
        r = m + jnp.log2(s)


def kernel(x):
    b, n, _ = x.shape
    return pl.pallas_call(
        _sinkhorn_body,
        grid=(b,),
        in_specs=[pl.BlockSpec((1, n, n), lambda i: (i, 0, 0))],
        out_specs=pl.BlockSpec((1, n, n), lambda i: (i, 0, 0)),
        out_shape=jax.ShapeDtypeStruct(x.shape, x.dtype),
        compiler_params=pltpu.CompilerParams(
            dimension_semantics=("parallel",),
        ),
    )(x)


# speedup vs baseline: 3.4124x; 1.0171x over previous
import jax
import jax.numpy as jnp
from jax.experimental import pallas as pl
from jax.experimental.pallas import tpu as pltpu

_SINKHORN_ITERS = 5
_TAU = 0.01
_EPS = 1e-6
_LOG2E = 1.4426950408889634
_BS = 2


def _sinkhorn_body(x_ref, o_ref):
    y0s = [x_ref[k] * (_LOG2E / _TAU) for k in range(_BS)]

    rs = []
    for y0 in y0s:
        m = jnp.max(y0, axis=1, keepdims=True)
        s = jnp.sum(jnp.exp2(y0 - m), axis=1, keepdims=True)
        rs.append(m + jnp.log2(s))

    for it in range(_SINKHORN_ITERS):
        last = it == _SINKHORN_ITERS - 1
        cs = []
        for k in range(_BS):
            u = y0s[k] - rs[k]
            m = jnp.max(u, axis=0, keepdims=True)
            e = jnp.exp2(u - m)
            s = jnp.sum(e, axis=0, keepdims=True)
            if last:
                o_ref[k] = e * (1.0 / s) + _EPS
            else:
                cs.append(m + jnp.log2(s))
        if last:
            break
        rs = []
        for k in range(_BS):
            t = y0s[k] - cs[k]
            m = jnp.max(t, axis=1, keepdims=True)
            s = jnp.sum(jnp.exp2(t - m), axis=1, keepdims=True)
            rs.append(m + jnp.log2(s))


def kernel(x):
    b, n, _ = x.shape
    return pl.pallas_call(
        _sinkhorn_body,
        grid=(b // _BS,),
        in_specs=[pl.BlockSpec((_BS, n, n), lambda i: (i, 0, 0))],
        out_specs=pl.BlockSpec((_BS, n, n), lambda i: (i, 0, 0)),
        out_shape=jax.ShapeDtypeStruct(x.shape, x.dtype),
        compiler_params=pltpu.CompilerParams(
            dimension_semantics=("parallel",),
        ),
    )(x)
